# baseline (speedup 1.0000x reference)
import functools

import jax
import jax.numpy as jnp
from jax import lax
from jax.experimental import pallas as pl
from jax.experimental.pallas import tpu as pltpu
from jax.experimental.pallas import tpu_sc as plsc

N = 10000
E = 320000
FIN = 128
HID = 128
HH = HID // 2
C = 40
CP = 48

NC = 2
NS = 16
NW = NC * NS
K = 80
NJ = E // NW // K
DEGW = 16
DEPTH = 5
L = 16

RB = 1000


def _mesh():
    return plsc.VectorSubcoreMesh(core_axis_name="c", subcore_axis_name="s")


SLAG = 2


def _pipelined_agg(tab, acc, idx_row, dst_row, rows, gsems, ssems, njt):
    for t in range(DEPTH):
        pltpu.async_copy(tab.at[idx_row(t)], rows.at[t], gsems[t])

    def body(jb, carry):
        for t in range(DEPTH):
            j = jb * DEPTH + t
            pltpu.make_async_copy(tab.at[idx_row(j)], rows.at[t], gsems[t]).wait()
            pltpu.async_copy(rows.at[t], acc.at[dst_row(j)], ssems[t], add=True)

            jp = j - SLAG
            b = (t - SLAG) % DEPTH

            @pl.when(jp >= 0)
            def _():
                pltpu.make_async_copy(rows.at[b], acc.at[dst_row(jp)],
                                      ssems[b]).wait()

                @pl.when(jp + DEPTH < njt)
                def _():
                    pltpu.async_copy(tab.at[idx_row(jp + DEPTH)], rows.at[b],
                                     gsems[b])
        return carry

    lax.fori_loop(0, njt // DEPTH, body, 0)
    for q in range(SLAG):
        j = njt - SLAG + q
        b = j % DEPTH
        pltpu.make_async_copy(rows.at[b], acc.at[dst_row(j)], ssems[b]).wait()


@functools.partial(
    pl.kernel,
    out_type=jax.ShapeDtypeStruct((N, FIN), jnp.float32),
    mesh=_mesh(),
    compiler_params=pltpu.CompilerParams(use_tc_tiling_on_sc=False),
    scratch_types=[
        pltpu.VMEM((NJ, K), jnp.int32),
        pltpu.VMEM((K, DEGW), jnp.float32),
        pltpu.VMEM_SHARED((N, DEGW), jnp.float32),
    ],
)
def _deg(eib_hbm, ones_hbm, zeros_hbm, out_hbm, dst_v, ones_v, acc):
    c = lax.axis_index("c")
    s = lax.axis_index("s")
    w = c * NS + s

    @pl.when(s == 0)
    def _():
        pltpu.sync_copy(zeros_hbm, acc)

    pltpu.sync_copy(ones_hbm, ones_v)
    pltpu.sync_copy(eib_hbm.at[1, w], dst_v)
    plsc.subcore_barrier()

    def body(j, carry):
        pltpu.sync_copy(ones_v, acc.at[dst_v.at[j]], add=True)
        return carry

    lax.fori_loop(0, NJ, body, 0)
    plsc.subcore_barrier()

    @pl.when(s == 0)
    def _():
        pltpu.sync_copy(acc, out_hbm.at[:, pl.ds(DEGW * c, DEGW)])


@functools.partial(
    pl.kernel,
    out_type=jax.ShapeDtypeStruct((N, FIN), jnp.float32),
    mesh=_mesh(),
    compiler_params=pltpu.CompilerParams(use_tc_tiling_on_sc=False),
    scratch_types=[
        pltpu.VMEM((2, NJ, K), jnp.int32),
        pltpu.VMEM((2, NJ, K), jnp.int32),
        pltpu.VMEM((DEPTH, K, HH), jnp.float32),
        pltpu.VMEM_SHARED((N, HH), jnp.float32),
        [pltpu.SemaphoreType.DMA] * DEPTH,
        [pltpu.SemaphoreType.DMA] * DEPTH,
    ],
)
def _agg_split(g1v_hbm, eib_hbm, zeros_hbm, out_hbm,
               src_v, dst_v, rows, acc, gsems, ssems):
    c = lax.axis_index("c")
    s = lax.axis_index("s")

    @pl.when(s == 0)
    def _():
        pltpu.sync_copy(zeros_hbm, acc)

    pltpu.sync_copy(eib_hbm.at[0, pl.ds(2 * s, 2)], src_v)
    pltpu.sync_copy(eib_hbm.at[1, pl.ds(2 * s, 2)], dst_v)

    for q in range(2):
        def dbl(j, carry):
            for t in range(K // L):
                v = src_v[q, j, pl.ds(t * L, L)]
                src_v[q, j, pl.ds(t * L, L)] = v + v + c
            return carry

        lax.fori_loop(0, NJ, dbl, 0)
    plsc.subcore_barrier()

    for q in range(2):
        _pipelined_agg(g1v_hbm, acc,
                       lambda j: src_v.at[q, j], lambda j: dst_v.at[q, j],
                       rows, gsems, ssems, NJ)
    plsc.subcore_barrier()

    @pl.when(s == 0)
    def _():
        pltpu.sync_copy(acc, out_hbm.at[:, pl.ds(HH * c, HH)])


@functools.partial(
    pl.kernel,
    out_type=jax.ShapeDtypeStruct((N, FIN), jnp.float32),
    mesh=_mesh(),
    compiler_params=pltpu.CompilerParams(use_tc_tiling_on_sc=False),
    scratch_types=[
        pltpu.VMEM((NJ, K), jnp.int32),
        pltpu.VMEM((NJ, K), jnp.int32),
        pltpu.VMEM((DEPTH, K, CP), jnp.float32),
        pltpu.VMEM_SHARED((N, CP), jnp.float32),
        [pltpu.SemaphoreType.DMA] * DEPTH,
        [pltpu.SemaphoreType.DMA] * DEPTH,
    ],
)
def _agg_full(g_hbm, eib_hbm, zeros_hbm, out_hbm,
              src_v, dst_v, rows, acc, gsems, ssems):
    c = lax.axis_index("c")
    s = lax.axis_index("s")
    w = c * NS + s

    @pl.when(s == 0)
    def _():
        pltpu.sync_copy(zeros_hbm, acc)

    pltpu.sync_copy(eib_hbm.at[0, w], src_v)
    pltpu.sync_copy(eib_hbm.at[1, w], dst_v)
    plsc.subcore_barrier()

    _pipelined_agg(g_hbm, acc,
                   lambda j: src_v.at[j], lambda j: dst_v.at[j],
                   rows, gsems, ssems, NJ)
    plsc.subcore_barrier()

    @pl.when(s == 0)
    def _():
        pltpu.sync_copy(acc, out_hbm.at[:, pl.ds(CP * c, CP)])


def _dinv_of(dg_ref):
    d = dg_ref[:, 0:1] + dg_ref[:, 16:17] + 1.0
    return lax.rsqrt(jnp.maximum(d, 1.0))


def _mm1_body(x_ref, w_ref, dg_ref, o_ref):
    dinv = _dinv_of(dg_ref)
    o_ref[...] = jnp.dot(x_ref[...], w_ref[...],
                         preferred_element_type=jnp.float32) * dinv


def _mid_body(s1_ref, g1_ref, dg_ref, b1_ref, w2_ref, o_ref):
    dinv = _dinv_of(dg_ref)
    z = (s1_ref[...] + g1_ref[...]) * dinv + b1_ref[...]
    h = jnp.maximum(z, 0.0)
    o_ref[...] = jnp.dot(h, w2_ref[...],
                         preferred_element_type=jnp.float32) * dinv


def _fin_body(s2_ref, g2_ref, dg_ref, b2_ref, o_ref):
    dinv = _dinv_of(dg_ref)
    z = (s2_ref[:, :CP] + s2_ref[:, CP:2 * CP] + g2_ref[...]) * dinv \
        + b2_ref[...]
    col = lax.broadcasted_iota(jnp.int32, (RB, CP), 1)
    valid = col < C
    zm = jnp.where(valid, z, -jnp.inf)
    m = jnp.max(zm, axis=1, keepdims=True)
    e = jnp.where(valid, jnp.exp(z - m), 0.0)
    ssum = jnp.sum(e, axis=1, keepdims=True)
    o_ref[...] = (z - m - jnp.log(ssum))[:, :C]


def _row_spec(width):
    return pl.BlockSpec((RB, width), lambda i: (i, 0))


def _const_spec(shape):
    return pl.BlockSpec(shape, lambda i: (0,) * len(shape))


@jax.jit
def kernel(x, edge_index, W1, b1, W2, b2):
    eib = edge_index.reshape(2, NW, NJ, K)
    ones16 = jnp.ones((K, DEGW), jnp.float32)
    zeros16 = jnp.zeros((N, DEGW), jnp.float32)
    zeros64 = jnp.zeros((N, HH), jnp.float32)
    zeros48 = jnp.zeros((N, CP), jnp.float32)
    W2p = jnp.pad(W2, ((0, 0), (0, CP - C)))
    b1r = b1.reshape(1, HID)
    b2r = jnp.pad(b2, (0, CP - C)).reshape(1, CP)

    dga = _deg(eib, ones16, zeros16)
    dg_spec = _row_spec(FIN)

    g1 = pl.pallas_call(
        _mm1_body,
        grid=(N // RB,),
        in_specs=[_row_spec(FIN), _const_spec((FIN, HID)), dg_spec],
        out_specs=_row_spec(HID),
        out_shape=jax.ShapeDtypeStruct((N, HID), jnp.float32),
    )(x, W1, dga)

    s1 = _agg_split(g1.reshape(2 * N, HH), eib, zeros64)

    g2 = pl.pallas_call(
        _mid_body,
        grid=(N // RB,),
        in_specs=[_row_spec(HID), _row_spec(HID), dg_spec,
                  _const_spec((1, HID)), _const_spec((HID, CP))],
        out_specs=_row_spec(CP),
        out_shape=jax.ShapeDtypeStruct((N, CP), jnp.float32),
    )(s1, g1, dga, b1r, W2p)

    s2 = _agg_full(g2, eib, zeros48)

    out = pl.pallas_call(
        _fin_body,
        grid=(N // RB,),
        in_specs=[_row_spec(FIN), _row_spec(CP), dg_spec,
                  _const_spec((1, CP))],
        out_specs=_row_spec(C),
        out_shape=jax.ShapeDtypeStruct((N, C), jnp.float32),
    )(s2, g2, dga, b2r)

    return out


# speedup vs baseline: 1.0821x; 1.0821x over previous
import functools

import jax
import jax.numpy as jnp
from jax import lax
from jax.experimental import pallas as pl
from jax.experimental.pallas import tpu as pltpu
from jax.experimental.pallas import tpu_sc as plsc

N = 10000
E = 320000
FIN = 128
HID = 128
HH = HID // 2
C = 40
CP = 48

NC = 2
NS = 16
NW = NC * NS
K = 80
NJ = E // NW // K
DEGW = 16
DEPTH = 5
L = 16

RB = 1000


def _mesh():
    return plsc.VectorSubcoreMesh(core_axis_name="c", subcore_axis_name="s")


def _pipelined_agg(tab, acc, idx_row, dst_row, rows, gsems, ssems, njt):
    del ssems
    for t in range(DEPTH):
        pltpu.async_copy(tab.at[idx_row(t)], rows.at[t], gsems[t])

    def body(jb, carry):
        for t in range(DEPTH):
            j = jb * DEPTH + t
            pltpu.make_async_copy(tab.at[idx_row(j)], rows.at[t], gsems[t]).wait()
            pltpu.sync_copy(rows.at[t], acc.at[dst_row(j)], add=True)

            @pl.when(j + DEPTH < njt)
            def _():
                pltpu.async_copy(tab.at[idx_row(j + DEPTH)], rows.at[t], gsems[t])
        return carry

    lax.fori_loop(0, njt // DEPTH, body, 0)


@functools.partial(
    pl.kernel,
    out_type=jax.ShapeDtypeStruct((N, FIN), jnp.float32),
    mesh=_mesh(),
    compiler_params=pltpu.CompilerParams(use_tc_tiling_on_sc=False),
    scratch_types=[
        pltpu.VMEM((NJ, K), jnp.int32),
        pltpu.VMEM((K, DEGW), jnp.float32),
        pltpu.VMEM_SHARED((N, DEGW), jnp.float32),
    ],
)
def _deg(eib_hbm, ones_hbm, zeros_hbm, out_hbm, dst_v, ones_v, acc):
    c = lax.axis_index("c")
    s = lax.axis_index("s")
    w = c * NS + s

    @pl.when(s == 0)
    def _():
        pltpu.sync_copy(zeros_hbm, acc)

    pltpu.sync_copy(ones_hbm, ones_v)
    pltpu.sync_copy(eib_hbm.at[1, w], dst_v)
    plsc.subcore_barrier()

    def body(j, carry):
        pltpu.sync_copy(ones_v, acc.at[dst_v.at[j]], add=True)
        return carry

    lax.fori_loop(0, NJ, body, 0)
    plsc.subcore_barrier()

    @pl.when(s == 0)
    def _():
        pltpu.sync_copy(acc, out_hbm.at[:, pl.ds(DEGW * c, DEGW)])


@functools.partial(
    pl.kernel,
    out_type=jax.ShapeDtypeStruct((N, FIN), jnp.float32),
    mesh=_mesh(),
    compiler_params=pltpu.CompilerParams(use_tc_tiling_on_sc=False),
    scratch_types=[
        pltpu.VMEM((2, NJ, K), jnp.int32),
        pltpu.VMEM((2, NJ, K), jnp.int32),
        pltpu.VMEM((DEPTH, K, HH), jnp.float32),
        pltpu.VMEM_SHARED((N, HH), jnp.float32),
        [pltpu.SemaphoreType.DMA] * DEPTH,
        [pltpu.SemaphoreType.DMA] * DEPTH,
    ],
)
def _agg_split(g1v_hbm, eib_hbm, zeros_hbm, out_hbm,
               src_v, dst_v, rows, acc, gsems, ssems):
    c = lax.axis_index("c")
    s = lax.axis_index("s")

    @pl.when(s == 0)
    def _():
        pltpu.sync_copy(zeros_hbm, acc)

    pltpu.sync_copy(eib_hbm.at[0, pl.ds(2 * s, 2)], src_v)
    pltpu.sync_copy(eib_hbm.at[1, pl.ds(2 * s, 2)], dst_v)

    for q in range(2):
        def dbl(j, carry):
            for t in range(K // L):
                v = src_v[q, j, pl.ds(t * L, L)]
                src_v[q, j, pl.ds(t * L, L)] = v + v + c
            return carry

        lax.fori_loop(0, NJ, dbl, 0)
    plsc.subcore_barrier()

    for q in range(2):
        _pipelined_agg(g1v_hbm, acc,
                       lambda j: src_v.at[q, j], lambda j: dst_v.at[q, j],
                       rows, gsems, ssems, NJ)
    plsc.subcore_barrier()

    @pl.when(s == 0)
    def _():
        pltpu.sync_copy(acc, out_hbm.at[:, pl.ds(HH * c, HH)])


@functools.partial(
    pl.kernel,
    out_type=jax.ShapeDtypeStruct((N, FIN), jnp.float32),
    mesh=_mesh(),
    compiler_params=pltpu.CompilerParams(use_tc_tiling_on_sc=False),
    scratch_types=[
        pltpu.VMEM((NJ, K), jnp.int32),
        pltpu.VMEM((NJ, K), jnp.int32),
        pltpu.VMEM((DEPTH, K, CP), jnp.float32),
        pltpu.VMEM_SHARED((N, CP), jnp.float32),
        [pltpu.SemaphoreType.DMA] * DEPTH,
        [pltpu.SemaphoreType.DMA] * DEPTH,
    ],
)
def _agg_full(g_hbm, eib_hbm, zeros_hbm, out_hbm,
              src_v, dst_v, rows, acc, gsems, ssems):
    c = lax.axis_index("c")
    s = lax.axis_index("s")
    w = c * NS + s

    @pl.when(s == 0)
    def _():
        pltpu.sync_copy(zeros_hbm, acc)

    pltpu.sync_copy(eib_hbm.at[0, w], src_v)
    pltpu.sync_copy(eib_hbm.at[1, w], dst_v)
    plsc.subcore_barrier()

    _pipelined_agg(g_hbm, acc,
                   lambda j: src_v.at[j], lambda j: dst_v.at[j],
                   rows, gsems, ssems, NJ)
    plsc.subcore_barrier()

    @pl.when(s == 0)
    def _():
        pltpu.sync_copy(acc, out_hbm.at[:, pl.ds(CP * c, CP)])


def _dinv_of(dg_ref):
    d = dg_ref[:, 0:1] + dg_ref[:, 16:17] + 1.0
    return lax.rsqrt(jnp.maximum(d, 1.0))


def _mm1_body(x_ref, w_ref, dg_ref, o_ref):
    dinv = _dinv_of(dg_ref)
    o_ref[...] = jnp.dot(x_ref[...], w_ref[...],
                         preferred_element_type=jnp.float32) * dinv


def _mid_body(s1_ref, g1_ref, dg_ref, b1_ref, w2_ref, o_ref):
    dinv = _dinv_of(dg_ref)
    z = (s1_ref[...] + g1_ref[...]) * dinv + b1_ref[...]
    h = jnp.maximum(z, 0.0)
    o_ref[...] = jnp.dot(h, w2_ref[...],
                         preferred_element_type=jnp.float32) * dinv


def _fin_body(s2_ref, g2_ref, dg_ref, b2_ref, o_ref):
    dinv = _dinv_of(dg_ref)
    z = (s2_ref[:, :CP] + s2_ref[:, CP:2 * CP] + g2_ref[...]) * dinv \
        + b2_ref[...]
    col = lax.broadcasted_iota(jnp.int32, (RB, CP), 1)
    valid = col < C
    zm = jnp.where(valid, z, -jnp.inf)
    m = jnp.max(zm, axis=1, keepdims=True)
    e = jnp.where(valid, jnp.exp(z - m), 0.0)
    ssum = jnp.sum(e, axis=1, keepdims=True)
    o_ref[...] = (z - m - jnp.log(ssum))[:, :C]


def _row_spec(width):
    return pl.BlockSpec((RB, width), lambda i: (i, 0))


def _const_spec(shape):
    return pl.BlockSpec(shape, lambda i: (0,) * len(shape))


@jax.jit
def kernel(x, edge_index, W1, b1, W2, b2):
    eib = edge_index.reshape(2, NW, NJ, K)
    ones16 = jnp.ones((K, DEGW), jnp.float32)
    zeros16 = jnp.zeros((N, DEGW), jnp.float32)
    zeros64 = jnp.zeros((N, HH), jnp.float32)
    zeros48 = jnp.zeros((N, CP), jnp.float32)
    W2p = jnp.pad(W2, ((0, 0), (0, CP - C)))
    b1r = b1.reshape(1, HID)
    b2r = jnp.pad(b2, (0, CP - C)).reshape(1, CP)

    dga = _deg(eib, ones16, zeros16)
    dg_spec = _row_spec(FIN)

    g1 = pl.pallas_call(
        _mm1_body,
        grid=(N // RB,),
        in_specs=[_row_spec(FIN), _const_spec((FIN, HID)), dg_spec],
        out_specs=_row_spec(HID),
        out_shape=jax.ShapeDtypeStruct((N, HID), jnp.float32),
    )(x, W1, dga)

    s1 = _agg_split(g1.reshape(2 * N, HH), eib, zeros64)

    g2 = pl.pallas_call(
        _mid_body,
        grid=(N // RB,),
        in_specs=[_row_spec(HID), _row_spec(HID), dg_spec,
                  _const_spec((1, HID)), _const_spec((HID, CP))],
        out_specs=_row_spec(CP),
        out_shape=jax.ShapeDtypeStruct((N, CP), jnp.float32),
    )(s1, g1, dga, b1r, W2p)

    s2 = _agg_full(g2, eib, zeros48)

    out = pl.pallas_call(
        _fin_body,
        grid=(N // RB,),
        in_specs=[_row_spec(FIN), _row_spec(CP), dg_spec,
                  _const_spec((1, CP))],
        out_specs=_row_spec(C),
        out_shape=jax.ShapeDtypeStruct((N, C), jnp.float32),
    )(s2, g2, dga, b2r)

    return out


# speedup vs baseline: 1.1129x; 1.0284x over previous
import functools

import jax
import jax.numpy as jnp
from jax import lax
from jax.experimental import pallas as pl
from jax.experimental.pallas import tpu as pltpu
from jax.experimental.pallas import tpu_sc as plsc

N = 10000
E = 320000
FIN = 128
HID = 128
HH = HID // 2
C = 40
CP = 48

NC = 2
NS = 16
NW = NC * NS
K = 80
NJ = E // NW // K
DEGW = 16
DEPTH = 5
DDEPTH = 5
L = 16

RB = 1000


def _mesh():
    return plsc.VectorSubcoreMesh(core_axis_name="c", subcore_axis_name="s")


def _pipelined_agg(tab, acc, idx_row, dst_row, rows, gsems, ssems, njt):
    del ssems
    for t in range(DEPTH):
        pltpu.async_copy(tab.at[idx_row(t)], rows.at[t], gsems[t])

    def body(jb, carry):
        for t in range(DEPTH):
            j = jb * DEPTH + t
            pltpu.make_async_copy(tab.at[idx_row(j)], rows.at[t], gsems[t]).wait()
            pltpu.sync_copy(rows.at[t], acc.at[dst_row(j)], add=True)

            @pl.when(j + DEPTH < njt)
            def _():
                pltpu.async_copy(tab.at[idx_row(j + DEPTH)], rows.at[t], gsems[t])
        return carry

    lax.fori_loop(0, njt // DEPTH, body, 0)


@functools.partial(
    pl.kernel,
    out_type=jax.ShapeDtypeStruct((N, FIN), jnp.float32),
    mesh=_mesh(),
    compiler_params=pltpu.CompilerParams(use_tc_tiling_on_sc=False),
    scratch_types=[
        pltpu.VMEM((NJ, K), jnp.int32),
        pltpu.VMEM((K, DEGW), jnp.float32),
        pltpu.VMEM_SHARED((N, DEGW), jnp.float32),
        [pltpu.SemaphoreType.DMA] * DDEPTH,
    ],
)
def _deg(eib_hbm, ones_hbm, zeros_hbm, out_hbm, dst_v, ones_v, acc, sems):
    c = lax.axis_index("c")
    s = lax.axis_index("s")
    w = c * NS + s

    @pl.when(s == 0)
    def _():
        pltpu.sync_copy(zeros_hbm, acc)

    pltpu.sync_copy(ones_hbm, ones_v)
    pltpu.sync_copy(eib_hbm.at[1, w], dst_v)
    plsc.subcore_barrier()

    def body(j, carry):
        for t in range(DDEPTH):
            jj = j * DDEPTH + t
            pltpu.async_copy(ones_v, acc.at[dst_v.at[jj]], sems[t], add=True)

            @pl.when(jj >= DDEPTH)
            def _():
                pltpu.make_async_copy(ones_v, acc.at[dst_v.at[jj - DDEPTH]],
                                      sems[t]).wait()
        return carry

    lax.fori_loop(0, NJ // DDEPTH, body, 0)
    for t in range(DDEPTH):
        j = NJ - DDEPTH + t
        pltpu.make_async_copy(ones_v, acc.at[dst_v.at[j]],
                              sems[j % DDEPTH]).wait()
    plsc.subcore_barrier()

    @pl.when(s == 0)
    def _():
        pltpu.sync_copy(acc, out_hbm.at[:, pl.ds(DEGW * c, DEGW)])


@functools.partial(
    pl.kernel,
    out_type=jax.ShapeDtypeStruct((N, FIN), jnp.float32),
    mesh=_mesh(),
    compiler_params=pltpu.CompilerParams(use_tc_tiling_on_sc=False),
    scratch_types=[
        pltpu.VMEM((2, NJ, K), jnp.int32),
        pltpu.VMEM((2, NJ, K), jnp.int32),
        pltpu.VMEM((DEPTH, K, HH), jnp.float32),
        pltpu.VMEM_SHARED((N, HH), jnp.float32),
        [pltpu.SemaphoreType.DMA] * DEPTH,
        [pltpu.SemaphoreType.DMA] * DEPTH,
    ],
)
def _agg_split(g1v_hbm, eib_hbm, zeros_hbm, out_hbm,
               src_v, dst_v, rows, acc, gsems, ssems):
    c = lax.axis_index("c")
    s = lax.axis_index("s")

    @pl.when(s == 0)
    def _():
        pltpu.sync_copy(zeros_hbm, acc)

    pltpu.sync_copy(eib_hbm.at[0, pl.ds(2 * s, 2)], src_v)
    pltpu.sync_copy(eib_hbm.at[1, pl.ds(2 * s, 2)], dst_v)

    for q in range(2):
        def dbl(j, carry):
            for t in range(K // L):
                v = src_v[q, j, pl.ds(t * L, L)]
                src_v[q, j, pl.ds(t * L, L)] = v + v + c
            return carry

        lax.fori_loop(0, NJ, dbl, 0)
    plsc.subcore_barrier()

    for q in range(2):
        _pipelined_agg(g1v_hbm, acc,
                       lambda j: src_v.at[q, j], lambda j: dst_v.at[q, j],
                       rows, gsems, ssems, NJ)
    plsc.subcore_barrier()

    @pl.when(s == 0)
    def _():
        pltpu.sync_copy(acc, out_hbm.at[:, pl.ds(HH * c, HH)])


@functools.partial(
    pl.kernel,
    out_type=jax.ShapeDtypeStruct((N, FIN), jnp.float32),
    mesh=_mesh(),
    compiler_params=pltpu.CompilerParams(use_tc_tiling_on_sc=False),
    scratch_types=[
        pltpu.VMEM((NJ, K), jnp.int32),
        pltpu.VMEM((NJ, K), jnp.int32),
        pltpu.VMEM((DEPTH, K, CP), jnp.float32),
        pltpu.VMEM_SHARED((N, CP), jnp.float32),
        [pltpu.SemaphoreType.DMA] * DEPTH,
        [pltpu.SemaphoreType.DMA] * DEPTH,
    ],
)
def _agg_full(g_hbm, eib_hbm, zeros_hbm, out_hbm,
              src_v, dst_v, rows, acc, gsems, ssems):
    c = lax.axis_index("c")
    s = lax.axis_index("s")
    w = c * NS + s

    @pl.when(s == 0)
    def _():
        pltpu.sync_copy(zeros_hbm, acc)

    pltpu.sync_copy(eib_hbm.at[0, w], src_v)
    pltpu.sync_copy(eib_hbm.at[1, w], dst_v)
    plsc.subcore_barrier()

    _pipelined_agg(g_hbm, acc,
                   lambda j: src_v.at[j], lambda j: dst_v.at[j],
                   rows, gsems, ssems, NJ)
    plsc.subcore_barrier()

    @pl.when(s == 0)
    def _():
        pltpu.sync_copy(acc, out_hbm.at[:, pl.ds(CP * c, CP)])


def _dinv_of(dg_ref):
    d = dg_ref[:, 0:1] + dg_ref[:, 16:17] + 1.0
    return lax.rsqrt(jnp.maximum(d, 1.0))


def _mm1_body(x_ref, w_ref, dg_ref, o_ref):
    dinv = _dinv_of(dg_ref)
    o_ref[...] = jnp.dot(x_ref[...], w_ref[...],
                         preferred_element_type=jnp.float32) * dinv


def _mid_body(s1_ref, g1_ref, dg_ref, b1_ref, w2_ref, o_ref):
    dinv = _dinv_of(dg_ref)
    z = (s1_ref[...] + g1_ref[...]) * dinv + b1_ref[...]
    h = jnp.maximum(z, 0.0)
    o_ref[...] = jnp.dot(h, w2_ref[...],
                         preferred_element_type=jnp.float32) * dinv


def _fin_body(s2_ref, g2_ref, dg_ref, b2_ref, o_ref):
    dinv = _dinv_of(dg_ref)
    z = (s2_ref[:, :CP] + s2_ref[:, CP:2 * CP] + g2_ref[...]) * dinv \
        + b2_ref[...]
    col = lax.broadcasted_iota(jnp.int32, (RB, CP), 1)
    valid = col < C
    zm = jnp.where(valid, z, -jnp.inf)
    m = jnp.max(zm, axis=1, keepdims=True)
    e = jnp.where(valid, jnp.exp(z - m), 0.0)
    ssum = jnp.sum(e, axis=1, keepdims=True)
    o_ref[...] = (z - m - jnp.log(ssum))[:, :C]


def _row_spec(width):
    return pl.BlockSpec((RB, width), lambda i: (i, 0))


def _const_spec(shape):
    return pl.BlockSpec(shape, lambda i: (0,) * len(shape))


@jax.jit
def kernel(x, edge_index, W1, b1, W2, b2):
    eib = edge_index.reshape(2, NW, NJ, K)
    ones16 = jnp.ones((K, DEGW), jnp.float32)
    zeros16 = jnp.zeros((N, DEGW), jnp.float32)
    zeros64 = jnp.zeros((N, HH), jnp.float32)
    zeros48 = jnp.zeros((N, CP), jnp.float32)
    W2p = jnp.pad(W2, ((0, 0), (0, CP - C)))
    b1r = b1.reshape(1, HID)
    b2r = jnp.pad(b2, (0, CP - C)).reshape(1, CP)

    dga = _deg(eib, ones16, zeros16)
    dg_spec = _row_spec(FIN)

    g1 = pl.pallas_call(
        _mm1_body,
        grid=(N // RB,),
        in_specs=[_row_spec(FIN), _const_spec((FIN, HID)), dg_spec],
        out_specs=_row_spec(HID),
        out_shape=jax.ShapeDtypeStruct((N, HID), jnp.float32),
    )(x, W1, dga)

    s1 = _agg_split(g1.reshape(2 * N, HH), eib, zeros64)

    g2 = pl.pallas_call(
        _mid_body,
        grid=(N // RB,),
        in_specs=[_row_spec(HID), _row_spec(HID), dg_spec,
                  _const_spec((1, HID)), _const_spec((HID, CP))],
        out_specs=_row_spec(CP),
        out_shape=jax.ShapeDtypeStruct((N, CP), jnp.float32),
    )(s1, g1, dga, b1r, W2p)

    s2 = _agg_full(g2, eib, zeros48)

    out = pl.pallas_call(
        _fin_body,
        grid=(N // RB,),
        in_specs=[_row_spec(FIN), _row_spec(CP), dg_spec,
                  _const_spec((1, CP))],
        out_specs=_row_spec(C),
        out_shape=jax.ShapeDtypeStruct((N, C), jnp.float32),
    )(s2, g2, dga, b2r)

    return out
